# Initial kernel scaffold; baseline (speedup 1.0000x reference)
import jax
import jax.numpy as jnp
from jax.experimental import pallas as pl


def kernel(x, edge_index, batch, edge_attr, params):
    raise NotImplementedError("write your pallas kernel here")


# speedup vs baseline: 1.7478x; 1.7478x over previous
import functools

import jax
import jax.numpy as jnp
from jax import lax
from jax.experimental import pallas as pl
from jax.experimental.pallas import tpu as pltpu
from jax.experimental.pallas import tpu_sc as plsc

N_NODES = 10000
HID = 16
EDGE_DIM = 4
N_GRAPHS = 64
ZCOLS = HID * HID + HID

NC = 2
NS = 16
BE = 128
EPW = 5120
E_PAD = NC * NS * EPW
N_ACC = 10112
STRIPE = N_ACC // NS

BN_BLK = 2000
BE_BLK = 2048


def _he_body(ea_ref, w_ref, b_ref, out_ref):
    a = ea_ref[...]
    for l in range(3):
        out_ref[l] = jnp.maximum(
            jnp.dot(a, w_ref[l], preferred_element_type=jnp.float32) + b_ref[l], 0.0)


def _he_call(ea_pad, ew1, eb1):
    return pl.pallas_call(
        _he_body,
        grid=(E_PAD // BE_BLK,),
        in_specs=[
            pl.BlockSpec((BE_BLK, EDGE_DIM), lambda i: (i, 0)),
            pl.BlockSpec((3, EDGE_DIM, HID), lambda i: (0, 0, 0)),
            pl.BlockSpec((3, 1, HID), lambda i: (0, 0, 0)),
        ],
        out_specs=pl.BlockSpec((3, BE_BLK, HID), lambda i: (0, i, 0)),
        out_shape=jax.ShapeDtypeStruct((3, E_PAD, HID), jnp.float32),
    )(ea_pad, ew1, eb1)


def _z_body(h_ref, c_ref, z_ref):
    z_ref[...] = jnp.dot(h_ref[...], c_ref[...], preferred_element_type=jnp.float32)


def _z_call(h, C):
    in_c = h.shape[-1]
    return pl.pallas_call(
        _z_body,
        grid=(N_NODES // BN_BLK,),
        in_specs=[
            pl.BlockSpec((BN_BLK, in_c), lambda i: (i, 0)),
            pl.BlockSpec((in_c, ZCOLS), lambda i: (0, 0)),
        ],
        out_specs=pl.BlockSpec((BN_BLK, ZCOLS), lambda i: (i, 0)),
        out_shape=jax.ShapeDtypeStruct((N_NODES, ZCOLS), jnp.float32),
    )(h, C)


@functools.lru_cache(maxsize=None)
def _sc_call(l):
    mesh = plsc.VectorSubcoreMesh(core_axis_name="c", subcore_axis_name="s")

    @functools.partial(
        pl.kernel,
        mesh=mesh,
        compiler_params=pltpu.CompilerParams(use_tc_tiling_on_sc=False),
        out_type=jax.ShapeDtypeStruct((NC, N_ACC, 32), jnp.float32),
        scratch_types=[
            pltpu.VMEM((BE,), jnp.int32),
            pltpu.VMEM((BE,), jnp.int32),
            pltpu.VMEM((BE, HID), jnp.float32),
            pltpu.VMEM((BE, ZCOLS), jnp.float32),
            pltpu.VMEM((BE, 32), jnp.float32),
            pltpu.VMEM_SHARED((N_ACC, 32), jnp.float32),
            pltpu.SemaphoreType.DMA,
        ],
    )
    def k(z_hbm, he_hbm, src_hbm, dst_hbm, zrow_hbm, out_hbm,
          src_v, dst_v, he_v, rows_v, msg_v, acc, sem):
        cid = lax.axis_index("c")
        sid = lax.axis_index("s")
        wid = cid * NS + sid

        pltpu.sync_copy(zrow_hbm, acc.at[pl.ds(sid * STRIPE, STRIPE)])

        lane = lax.iota(jnp.int32, 16)
        onevec = jnp.where(lane == 0, 1.0, 0.0).astype(jnp.float32)

        def fill(i, _):
            msg_v[i, pl.ds(HID, 16)] = onevec
            return 0
        lax.fori_loop(0, BE, fill, 0)

        plsc.subcore_barrier()

        def step(j, _):
            base = wid * EPW + j * BE
            pltpu.sync_copy(src_hbm.at[pl.ds(base, BE)], src_v)
            pltpu.sync_copy(dst_hbm.at[pl.ds(base, BE)], dst_v)
            pltpu.sync_copy(he_hbm.at[l, pl.ds(base, BE)], he_v)
            pltpu.async_copy(z_hbm.at[src_v], rows_v, sem).wait()

            def edge(e, _):
                hv = he_v[e, pl.ds(0, HID)]
                a = rows_v[e, pl.ds(HID * HID, 16)]
                for kk in range(HID):
                    a = a + hv[kk] * rows_v[e, pl.ds(HID * kk, 16)]
                msg_v[e, pl.ds(0, 16)] = a
                return 0
            lax.fori_loop(0, BE, edge, 0)

            pltpu.sync_copy(msg_v, acc.at[dst_v], add=True)
            return 0
        lax.fori_loop(0, EPW // BE, step, 0)

        plsc.subcore_barrier()
        pltpu.sync_copy(acc.at[pl.ds(sid * STRIPE, STRIPE)],
                        out_hbm.at[cid, pl.ds(sid * STRIPE, STRIPE)])

    return k


def _c1_body(p_ref, h_ref, root_ref, bias_ref, t_ref, st_ref):
    i = pl.program_id(0)
    p0 = p_ref[0]
    p1 = p_ref[1]
    s = p0[:, 0:HID] + p1[:, 0:HID]
    cnt = p0[:, HID:HID + 1] + p1[:, HID:HID + 1]
    agg = s / jnp.maximum(cnt, 1.0)
    t = agg + jnp.dot(h_ref[...], root_ref[...],
                      preferred_element_type=jnp.float32) + bias_ref[...]
    t_ref[...] = t
    blk = jnp.concatenate(
        [jnp.sum(t, axis=0, keepdims=True), jnp.sum(t * t, axis=0, keepdims=True)],
        axis=0)

    @pl.when(i == 0)
    def _():
        st_ref[...] = blk

    @pl.when(i > 0)
    def _():
        st_ref[...] = st_ref[...] + blk


def _c1_call(P, h, root, bias):
    in_c = h.shape[-1]
    return pl.pallas_call(
        _c1_body,
        grid=(N_NODES // BN_BLK,),
        in_specs=[
            pl.BlockSpec((NC, BN_BLK, 32), lambda i: (0, i, 0)),
            pl.BlockSpec((BN_BLK, in_c), lambda i: (i, 0)),
            pl.BlockSpec((in_c, HID), lambda i: (0, 0)),
            pl.BlockSpec((1, HID), lambda i: (0, 0)),
        ],
        out_specs=[
            pl.BlockSpec((BN_BLK, HID), lambda i: (i, 0)),
            pl.BlockSpec((2, HID), lambda i: (0, 0)),
        ],
        out_shape=[
            jax.ShapeDtypeStruct((N_NODES, HID), jnp.float32),
            jax.ShapeDtypeStruct((2, HID), jnp.float32),
        ],
    )(P, h, root, bias)


def _c2_body_res(t_ref, st_ref, g_ref, b_ref, hp_ref, o_ref):
    mean = st_ref[0:1, :] / N_NODES
    var = st_ref[1:2, :] / N_NODES - mean * mean
    xn = (t_ref[...] - mean) * lax.rsqrt(var + 1e-5) * g_ref[...] + b_ref[...]
    o_ref[...] = jnp.maximum(xn, 0.0) + hp_ref[...]


def _c2_body_nores(t_ref, st_ref, g_ref, b_ref, o_ref):
    mean = st_ref[0:1, :] / N_NODES
    var = st_ref[1:2, :] / N_NODES - mean * mean
    xn = (t_ref[...] - mean) * lax.rsqrt(var + 1e-5) * g_ref[...] + b_ref[...]
    o_ref[...] = jnp.maximum(xn, 0.0)


def _c2_call(residual, t, st, g, b, h_prev):
    in_specs = [
        pl.BlockSpec((BN_BLK, HID), lambda i: (i, 0)),
        pl.BlockSpec((2, HID), lambda i: (0, 0)),
        pl.BlockSpec((1, HID), lambda i: (0, 0)),
        pl.BlockSpec((1, HID), lambda i: (0, 0)),
    ]
    args = [t, st, g, b]
    if residual:
        in_specs.append(pl.BlockSpec((BN_BLK, HID), lambda i: (i, 0)))
        args.append(h_prev)
    return pl.pallas_call(
        _c2_body_res if residual else _c2_body_nores,
        grid=(N_NODES // BN_BLK,),
        in_specs=in_specs,
        out_specs=pl.BlockSpec((BN_BLK, HID), lambda i: (i, 0)),
        out_shape=jax.ShapeDtypeStruct((N_NODES, HID), jnp.float32),
    )(*args)


def _pool_body(h_ref, b_ref, w1_ref, b1_ref, w2_ref, b2_ref, out_ref,
               acc_s, cnt_s):
    i = pl.program_id(0)
    nsteps = pl.num_programs(0)
    bids = b_ref[:, 0]
    oh = (bids[:, None] == lax.broadcasted_iota(jnp.int32, (1, N_GRAPHS), 1)
          ).astype(jnp.float32)
    ssum = lax.dot_general(oh, h_ref[...], (((0,), (0,)), ((), ())),
                           preferred_element_type=jnp.float32)
    scnt = jnp.sum(oh, axis=0, keepdims=True)

    @pl.when(i == 0)
    def _():
        acc_s[...] = ssum
        cnt_s[...] = scnt

    @pl.when(i > 0)
    def _():
        acc_s[...] = acc_s[...] + ssum
        cnt_s[...] = cnt_s[...] + scnt

    @pl.when(i == nsteps - 1)
    def _():
        pooled = acc_s[...] / jnp.maximum(jnp.reshape(cnt_s[...], (N_GRAPHS, 1)), 1.0)
        h1 = jnp.maximum(
            jnp.dot(pooled, w1_ref[...], preferred_element_type=jnp.float32)
            + b1_ref[...], 0.0)
        out_ref[...] = (jnp.dot(h1, w2_ref[...], preferred_element_type=jnp.float32)
                        + b2_ref[...])


def _pool_call(h, batch2d, w1, b1, w2, b2):
    return pl.pallas_call(
        _pool_body,
        grid=(N_NODES // BN_BLK,),
        in_specs=[
            pl.BlockSpec((BN_BLK, HID), lambda i: (i, 0)),
            pl.BlockSpec((BN_BLK, 1), lambda i: (i, 0)),
            pl.BlockSpec((HID, HID // 2), lambda i: (0, 0)),
            pl.BlockSpec((1, HID // 2), lambda i: (0, 0)),
            pl.BlockSpec((HID // 2, 1), lambda i: (0, 0)),
            pl.BlockSpec((1, 1), lambda i: (0, 0)),
        ],
        out_specs=pl.BlockSpec((N_GRAPHS, 1), lambda i: (0, 0)),
        out_shape=jax.ShapeDtypeStruct((N_GRAPHS, 1), jnp.float32),
        scratch_shapes=[
            pltpu.VMEM((N_GRAPHS, HID), jnp.float32),
            pltpu.VMEM((1, N_GRAPHS), jnp.float32),
        ],
    )(h, batch2d, w1, b1, w2, b2)


def kernel(x, edge_index, batch, edge_attr, params):
    E = edge_attr.shape[0]
    npad = E_PAD - E
    src_p = jnp.concatenate([edge_index[0], jnp.zeros((npad,), jnp.int32)])
    dst_p = jnp.concatenate([edge_index[1], jnp.full((npad,), N_NODES, jnp.int32)])
    ea_p = jnp.concatenate([edge_attr, jnp.zeros((npad, EDGE_DIM), jnp.float32)])

    ew1 = jnp.stack([p["ew1"] for p in params["convs"]])
    eb1 = jnp.stack([p["eb1"] for p in params["convs"]])[:, None, :]
    he_all = _he_call(ea_p, ew1, eb1)
    zeros_stripe = jnp.zeros((STRIPE, 32), jnp.float32)

    h = x
    for l, p in enumerate(params["convs"]):
        in_c = h.shape[-1]
        T = p["ew2"].reshape(HID, in_c, HID).transpose(1, 0, 2).reshape(in_c, HID * HID)
        C = jnp.concatenate([T, p["eb2"].reshape(in_c, HID)], axis=1)
        Z = _z_call(h, C)
        P = _sc_call(l)(Z, he_all, src_p, dst_p, zeros_stripe)
        t, st = _c1_call(P, h, p["root"], p["bias"][None, :])
        h = _c2_call(l > 0, t, st, p["bn_g"][None, :], p["bn_b"][None, :], h)

    return _pool_call(h, batch[:, None], params["lin1_w"],
                      params["lin1_b"][None, :], params["lin2_w"],
                      params["lin2_b"][None, :])


# speedup vs baseline: 2.4797x; 1.4187x over previous
import functools

import jax
import jax.numpy as jnp
from jax import lax
from jax.experimental import pallas as pl
from jax.experimental.pallas import tpu as pltpu
from jax.experimental.pallas import tpu_sc as plsc

N_NODES = 10000
HID = 16
EDGE_DIM = 4
N_GRAPHS = 64
ZCOLS = HID * HID + HID

NC = 2
NS = 16
BE = 128
EPW = 5120
E_PAD = NC * NS * EPW
N_ACC = 10112
STRIPE = N_ACC // NS

BN_BLK = 2000
BE_BLK = 2048


def _he_body(ea_ref, w_ref, b_ref, out_ref):
    a = ea_ref[...]
    for l in range(3):
        out_ref[l] = jnp.maximum(
            jnp.dot(a, w_ref[l], preferred_element_type=jnp.float32) + b_ref[l], 0.0)


def _he_call(ea_pad, ew1, eb1):
    return pl.pallas_call(
        _he_body,
        grid=(E_PAD // BE_BLK,),
        in_specs=[
            pl.BlockSpec((BE_BLK, EDGE_DIM), lambda i: (i, 0)),
            pl.BlockSpec((3, EDGE_DIM, HID), lambda i: (0, 0, 0)),
            pl.BlockSpec((3, 1, HID), lambda i: (0, 0, 0)),
        ],
        out_specs=pl.BlockSpec((3, BE_BLK, HID), lambda i: (0, i, 0)),
        out_shape=jax.ShapeDtypeStruct((3, E_PAD, HID), jnp.float32),
    )(ea_pad, ew1, eb1)


def _z_body(h_ref, c_ref, z_ref):
    z_ref[...] = jnp.dot(h_ref[...], c_ref[...], preferred_element_type=jnp.float32)


def _z_call(h, C):
    in_c = h.shape[-1]
    return pl.pallas_call(
        _z_body,
        grid=(N_NODES // BN_BLK,),
        in_specs=[
            pl.BlockSpec((BN_BLK, in_c), lambda i: (i, 0)),
            pl.BlockSpec((in_c, ZCOLS), lambda i: (0, 0)),
        ],
        out_specs=pl.BlockSpec((BN_BLK, ZCOLS), lambda i: (i, 0)),
        out_shape=jax.ShapeDtypeStruct((N_NODES, ZCOLS), jnp.float32),
    )(h, C)


NB = EPW // BE


@functools.lru_cache(maxsize=None)
def _sc_call(l):
    mesh = plsc.VectorSubcoreMesh(core_axis_name="c", subcore_axis_name="s")

    @functools.partial(
        pl.kernel,
        mesh=mesh,
        compiler_params=pltpu.CompilerParams(use_tc_tiling_on_sc=False),
        out_type=jax.ShapeDtypeStruct((NC, N_ACC, 32), jnp.float32),
        scratch_types=[
            pltpu.VMEM((NB, BE), jnp.int32),
            pltpu.VMEM((NB, BE), jnp.int32),
            pltpu.VMEM((BE, HID), jnp.float32),
            pltpu.VMEM((BE, HID), jnp.float32),
            pltpu.VMEM((BE, ZCOLS), jnp.float32),
            pltpu.VMEM((BE, ZCOLS), jnp.float32),
            pltpu.VMEM((BE, 32), jnp.float32),
            pltpu.VMEM_SHARED((N_ACC, 32), jnp.float32),
            pltpu.SemaphoreType.DMA,
            pltpu.SemaphoreType.DMA,
            pltpu.SemaphoreType.DMA,
            pltpu.SemaphoreType.DMA,
        ],
    )
    def k(z_hbm, he_hbm, src_hbm, dst_hbm, zrow_hbm, out_hbm,
          sidx_v, didx_v, he_v0, he_v1, rows_v0, rows_v1, msg_v, acc,
          gsem0, gsem1, hsem0, hsem1):
        cid = lax.axis_index("c")
        sid = lax.axis_index("s")
        wid = cid * NS + sid

        pltpu.sync_copy(zrow_hbm, acc.at[pl.ds(sid * STRIPE, STRIPE)])

        pltpu.sync_copy(src_hbm.at[pl.ds(wid * NB, NB)], sidx_v)
        pltpu.sync_copy(dst_hbm.at[pl.ds(wid * NB, NB)], didx_v)

        lane = lax.iota(jnp.int32, 16)
        onevec = jnp.where(lane == 0, 1.0, 0.0).astype(jnp.float32)

        def fill(i, _):
            msg_v[i, pl.ds(HID, 16)] = onevec
            return 0
        lax.fori_loop(0, BE, fill, 0)

        plsc.subcore_barrier()

        def issue(j, rows_v, he_v, gsem, hsem):
            pltpu.async_copy(z_hbm.at[sidx_v.at[j]], rows_v, gsem)
            pltpu.async_copy(he_hbm.at[l, pl.ds(wid * EPW + j * BE, BE)],
                             he_v, hsem)

        def wait(rows_v, he_v, gsem, hsem):
            pltpu.make_async_copy(z_hbm.at[sidx_v.at[0]], rows_v, gsem).wait()
            pltpu.make_async_copy(he_hbm.at[l, pl.ds(0, BE)], he_v, hsem).wait()

        def compute_scatter(j, rows_v, he_v):
            def edge(e, _):
                hv = he_v[e, pl.ds(0, HID)]
                accs = [rows_v[e, pl.ds(HID * HID, 16)],
                        hv[0] * rows_v[e, pl.ds(0, 16)],
                        hv[1] * rows_v[e, pl.ds(HID, 16)],
                        hv[2] * rows_v[e, pl.ds(2 * HID, 16)]]
                for kk in range(3, HID):
                    accs[kk % 4] = accs[kk % 4] + hv[kk] * rows_v[e, pl.ds(HID * kk, 16)]
                msg_v[e, pl.ds(0, 16)] = (accs[0] + accs[1]) + (accs[2] + accs[3])
                return 0
            lax.fori_loop(0, BE, edge, 0)
            pltpu.sync_copy(msg_v, acc.at[didx_v.at[j]], add=True)

        issue(0, rows_v0, he_v0, gsem0, hsem0)

        def pair(p, _):
            j = 2 * p
            issue(j + 1, rows_v1, he_v1, gsem1, hsem1)
            wait(rows_v0, he_v0, gsem0, hsem0)
            compute_scatter(j, rows_v0, he_v0)

            @pl.when(p < NB // 2 - 1)
            def _():
                issue(j + 2, rows_v0, he_v0, gsem0, hsem0)
            wait(rows_v1, he_v1, gsem1, hsem1)
            compute_scatter(j + 1, rows_v1, he_v1)
            return 0
        lax.fori_loop(0, NB // 2, pair, 0)

        plsc.subcore_barrier()
        pltpu.sync_copy(acc.at[pl.ds(sid * STRIPE, STRIPE)],
                        out_hbm.at[cid, pl.ds(sid * STRIPE, STRIPE)])

    return k


def _c1_body(p_ref, h_ref, root_ref, bias_ref, t_ref, st_ref):
    i = pl.program_id(0)
    p0 = p_ref[0]
    p1 = p_ref[1]
    s = p0[:, 0:HID] + p1[:, 0:HID]
    cnt = p0[:, HID:HID + 1] + p1[:, HID:HID + 1]
    agg = s / jnp.maximum(cnt, 1.0)
    t = agg + jnp.dot(h_ref[...], root_ref[...],
                      preferred_element_type=jnp.float32) + bias_ref[...]
    t_ref[...] = t
    blk = jnp.concatenate(
        [jnp.sum(t, axis=0, keepdims=True), jnp.sum(t * t, axis=0, keepdims=True)],
        axis=0)

    @pl.when(i == 0)
    def _():
        st_ref[...] = blk

    @pl.when(i > 0)
    def _():
        st_ref[...] = st_ref[...] + blk


def _c1_call(P, h, root, bias):
    in_c = h.shape[-1]
    return pl.pallas_call(
        _c1_body,
        grid=(N_NODES // BN_BLK,),
        in_specs=[
            pl.BlockSpec((NC, BN_BLK, 32), lambda i: (0, i, 0)),
            pl.BlockSpec((BN_BLK, in_c), lambda i: (i, 0)),
            pl.BlockSpec((in_c, HID), lambda i: (0, 0)),
            pl.BlockSpec((1, HID), lambda i: (0, 0)),
        ],
        out_specs=[
            pl.BlockSpec((BN_BLK, HID), lambda i: (i, 0)),
            pl.BlockSpec((2, HID), lambda i: (0, 0)),
        ],
        out_shape=[
            jax.ShapeDtypeStruct((N_NODES, HID), jnp.float32),
            jax.ShapeDtypeStruct((2, HID), jnp.float32),
        ],
    )(P, h, root, bias)


def _c2_body_res(t_ref, st_ref, g_ref, b_ref, hp_ref, o_ref):
    mean = st_ref[0:1, :] / N_NODES
    var = st_ref[1:2, :] / N_NODES - mean * mean
    xn = (t_ref[...] - mean) * lax.rsqrt(var + 1e-5) * g_ref[...] + b_ref[...]
    o_ref[...] = jnp.maximum(xn, 0.0) + hp_ref[...]


def _c2_body_nores(t_ref, st_ref, g_ref, b_ref, o_ref):
    mean = st_ref[0:1, :] / N_NODES
    var = st_ref[1:2, :] / N_NODES - mean * mean
    xn = (t_ref[...] - mean) * lax.rsqrt(var + 1e-5) * g_ref[...] + b_ref[...]
    o_ref[...] = jnp.maximum(xn, 0.0)


def _c2_call(residual, t, st, g, b, h_prev):
    in_specs = [
        pl.BlockSpec((BN_BLK, HID), lambda i: (i, 0)),
        pl.BlockSpec((2, HID), lambda i: (0, 0)),
        pl.BlockSpec((1, HID), lambda i: (0, 0)),
        pl.BlockSpec((1, HID), lambda i: (0, 0)),
    ]
    args = [t, st, g, b]
    if residual:
        in_specs.append(pl.BlockSpec((BN_BLK, HID), lambda i: (i, 0)))
        args.append(h_prev)
    return pl.pallas_call(
        _c2_body_res if residual else _c2_body_nores,
        grid=(N_NODES // BN_BLK,),
        in_specs=in_specs,
        out_specs=pl.BlockSpec((BN_BLK, HID), lambda i: (i, 0)),
        out_shape=jax.ShapeDtypeStruct((N_NODES, HID), jnp.float32),
    )(*args)


def _pool_body(h_ref, b_ref, w1_ref, b1_ref, w2_ref, b2_ref, out_ref,
               acc_s, cnt_s):
    i = pl.program_id(0)
    nsteps = pl.num_programs(0)
    bids = b_ref[:, 0]
    oh = (bids[:, None] == lax.broadcasted_iota(jnp.int32, (1, N_GRAPHS), 1)
          ).astype(jnp.float32)
    ssum = lax.dot_general(oh, h_ref[...], (((0,), (0,)), ((), ())),
                           preferred_element_type=jnp.float32)
    scnt = jnp.sum(oh, axis=0, keepdims=True)

    @pl.when(i == 0)
    def _():
        acc_s[...] = ssum
        cnt_s[...] = scnt

    @pl.when(i > 0)
    def _():
        acc_s[...] = acc_s[...] + ssum
        cnt_s[...] = cnt_s[...] + scnt

    @pl.when(i == nsteps - 1)
    def _():
        pooled = acc_s[...] / jnp.maximum(jnp.reshape(cnt_s[...], (N_GRAPHS, 1)), 1.0)
        h1 = jnp.maximum(
            jnp.dot(pooled, w1_ref[...], preferred_element_type=jnp.float32)
            + b1_ref[...], 0.0)
        out_ref[...] = (jnp.dot(h1, w2_ref[...], preferred_element_type=jnp.float32)
                        + b2_ref[...])


def _pool_call(h, batch2d, w1, b1, w2, b2):
    return pl.pallas_call(
        _pool_body,
        grid=(N_NODES // BN_BLK,),
        in_specs=[
            pl.BlockSpec((BN_BLK, HID), lambda i: (i, 0)),
            pl.BlockSpec((BN_BLK, 1), lambda i: (i, 0)),
            pl.BlockSpec((HID, HID // 2), lambda i: (0, 0)),
            pl.BlockSpec((1, HID // 2), lambda i: (0, 0)),
            pl.BlockSpec((HID // 2, 1), lambda i: (0, 0)),
            pl.BlockSpec((1, 1), lambda i: (0, 0)),
        ],
        out_specs=pl.BlockSpec((N_GRAPHS, 1), lambda i: (0, 0)),
        out_shape=jax.ShapeDtypeStruct((N_GRAPHS, 1), jnp.float32),
        scratch_shapes=[
            pltpu.VMEM((N_GRAPHS, HID), jnp.float32),
            pltpu.VMEM((1, N_GRAPHS), jnp.float32),
        ],
    )(h, batch2d, w1, b1, w2, b2)


def kernel(x, edge_index, batch, edge_attr, params):
    E = edge_attr.shape[0]
    npad = E_PAD - E
    src_p = jnp.concatenate(
        [edge_index[0], jnp.zeros((npad,), jnp.int32)]).reshape(E_PAD // BE, BE)
    dst_p = jnp.concatenate(
        [edge_index[1], jnp.full((npad,), N_NODES, jnp.int32)]).reshape(E_PAD // BE, BE)
    ea_p = jnp.concatenate([edge_attr, jnp.zeros((npad, EDGE_DIM), jnp.float32)])

    ew1 = jnp.stack([p["ew1"] for p in params["convs"]])
    eb1 = jnp.stack([p["eb1"] for p in params["convs"]])[:, None, :]
    he_all = _he_call(ea_p, ew1, eb1)
    zeros_stripe = jnp.zeros((STRIPE, 32), jnp.float32)

    h = x
    for l, p in enumerate(params["convs"]):
        in_c = h.shape[-1]
        T = p["ew2"].reshape(HID, in_c, HID).transpose(1, 0, 2).reshape(in_c, HID * HID)
        C = jnp.concatenate([T, p["eb2"].reshape(in_c, HID)], axis=1)
        Z = _z_call(h, C)
        P = _sc_call(l)(Z, he_all, src_p, dst_p, zeros_stripe)
        t, st = _c1_call(P, h, p["root"], p["bias"][None, :])
        h = _c2_call(l > 0, t, st, p["bn_g"][None, :], p["bn_b"][None, :], h)

    return _pool_call(h, batch[:, None], params["lin1_w"],
                      params["lin1_b"][None, :], params["lin2_w"],
                      params["lin2_b"][None, :])


# speedup vs baseline: 2.8311x; 1.1417x over previous
import functools

import jax
import jax.numpy as jnp
from jax import lax
from jax.experimental import pallas as pl
from jax.experimental.pallas import tpu as pltpu
from jax.experimental.pallas import tpu_sc as plsc

N_NODES = 10000
HID = 16
EDGE_DIM = 4
N_GRAPHS = 64
ZCOLS = HID * HID + HID

NC = 2
NS = 16
BE = 128
EPW = 5120
E_PAD = NC * NS * EPW
N_ACC = 10112
STRIPE = N_ACC // NS

BN_BLK = 2000


def _recip(x):
    r = 1.0 / x
    return r * (2.0 - x * r)


def _rsqrt_acc(x):
    r = lax.rsqrt(x)
    return 0.5 * r * (3.0 - x * r * r)


def _dot(a, b):
    return jnp.dot(a, b, preferred_element_type=jnp.float32,
                   precision=lax.Precision.HIGHEST)


def _bfdot(a, b):
    return jnp.dot(a.astype(jnp.bfloat16), b.astype(jnp.bfloat16),
                   preferred_element_type=jnp.float32)


def _z_body(h_ref, c_ref, z_ref):
    z_ref[...] = _dot(h_ref[...], c_ref[...])


def _z_call(h, C):
    in_c = h.shape[-1]
    return pl.pallas_call(
        _z_body,
        grid=(N_NODES // BN_BLK,),
        in_specs=[
            pl.BlockSpec((BN_BLK, in_c), lambda i: (i, 0)),
            pl.BlockSpec((in_c, ZCOLS), lambda i: (0, 0)),
        ],
        out_specs=pl.BlockSpec((BN_BLK, ZCOLS), lambda i: (i, 0)),
        out_shape=jax.ShapeDtypeStruct((N_NODES, ZCOLS), jnp.float32),
    )(h, C)


NB = EPW // BE


@functools.lru_cache(maxsize=None)
def _sc_call():
    mesh = plsc.VectorSubcoreMesh(core_axis_name="c", subcore_axis_name="s")

    @functools.partial(
        pl.kernel,
        mesh=mesh,
        compiler_params=pltpu.CompilerParams(use_tc_tiling_on_sc=False),
        out_type=jax.ShapeDtypeStruct((NC, N_ACC, 32), jnp.float32),
        scratch_types=[
            pltpu.VMEM((NB, BE), jnp.int32),
            pltpu.VMEM((NB, BE), jnp.int32),
            pltpu.VMEM((EPW * EDGE_DIM,), jnp.float32),
            pltpu.VMEM((EDGE_DIM, HID), jnp.float32),
            pltpu.VMEM((1, HID), jnp.float32),
            pltpu.VMEM((BE, ZCOLS), jnp.float32),
            pltpu.VMEM((BE, ZCOLS), jnp.float32),
            pltpu.VMEM((BE, 32), jnp.float32),
            pltpu.VMEM_SHARED((N_ACC, 32), jnp.float32),
            pltpu.SemaphoreType.DMA,
            pltpu.SemaphoreType.DMA,
        ],
    )
    def k(z_hbm, ea_hbm, ew1_hbm, eb1_hbm, src_hbm, dst_hbm, zrow_hbm, out_hbm,
          sidx_v, didx_v, ea_v, ew1_v, eb1_v, rows_v0, rows_v1, msg_v,
          acc, gsem0, gsem1):
        cid = lax.axis_index("c")
        sid = lax.axis_index("s")
        wid = cid * NS + sid

        pltpu.sync_copy(zrow_hbm, acc.at[pl.ds(sid * STRIPE, STRIPE)])

        pltpu.sync_copy(src_hbm.at[pl.ds(wid * NB, NB)], sidx_v)
        pltpu.sync_copy(dst_hbm.at[pl.ds(wid * NB, NB)], didx_v)
        pltpu.sync_copy(ea_hbm.at[pl.ds(wid * EPW * EDGE_DIM, EPW * EDGE_DIM)], ea_v)
        pltpu.sync_copy(ew1_hbm, ew1_v)
        pltpu.sync_copy(eb1_hbm, eb1_v)

        lane = lax.iota(jnp.int32, 16)
        onevec = jnp.where(lane == 0, 1.0, 0.0).astype(jnp.float32)

        def fill(i, _):
            msg_v[i, pl.ds(HID, 16)] = onevec
            return 0
        lax.fori_loop(0, BE, fill, 0)

        plsc.subcore_barrier()

        def issue(j, rows_v, gsem):
            pltpu.async_copy(z_hbm.at[sidx_v.at[j]], rows_v, gsem)

        def wait(rows_v, gsem):
            pltpu.make_async_copy(z_hbm.at[sidx_v.at[0]], rows_v, gsem).wait()

        def compute_scatter(j, rows_v):
            w0 = ew1_v[0, pl.ds(0, HID)]
            w1 = ew1_v[1, pl.ds(0, HID)]
            w2 = ew1_v[2, pl.ds(0, HID)]
            w3 = ew1_v[3, pl.ds(0, HID)]
            bv = eb1_v[0, pl.ds(0, HID)]

            def pairfn(m, _):
                av = ea_v[pl.ds(j * (BE * EDGE_DIM) + m * (2 * EDGE_DIM), 16)]
                for half in range(2):
                    e = 2 * m + half
                    o4 = half * EDGE_DIM
                    hv = ((((av[o4] * w0) + av[o4 + 1] * w1)
                           + av[o4 + 2] * w2) + av[o4 + 3] * w3) + bv
                    hv = jnp.maximum(hv, 0.0)
                    accs = [rows_v[e, pl.ds(HID * HID, 16)],
                            hv[0] * rows_v[e, pl.ds(0, 16)],
                            hv[1] * rows_v[e, pl.ds(HID, 16)],
                            hv[2] * rows_v[e, pl.ds(2 * HID, 16)]]
                    for kk in range(3, HID):
                        accs[kk % 4] = accs[kk % 4] + hv[kk] * rows_v[e, pl.ds(HID * kk, 16)]
                    msg_v[e, pl.ds(0, 16)] = (accs[0] + accs[1]) + (accs[2] + accs[3])
                return 0
            lax.fori_loop(0, BE // 2, pairfn, 0)
            pltpu.sync_copy(msg_v, acc.at[didx_v.at[j]], add=True)

        issue(0, rows_v0, gsem0)

        def pair(p, _):
            j = 2 * p
            issue(j + 1, rows_v1, gsem1)
            wait(rows_v0, gsem0)
            compute_scatter(j, rows_v0)

            @pl.when(p < NB // 2 - 1)
            def _():
                issue(j + 2, rows_v0, gsem0)
            wait(rows_v1, gsem1)
            compute_scatter(j + 1, rows_v1)
            return 0
        lax.fori_loop(0, NB // 2, pair, 0)

        plsc.subcore_barrier()
        pltpu.sync_copy(acc.at[pl.ds(sid * STRIPE, STRIPE)],
                        out_hbm.at[cid, pl.ds(sid * STRIPE, STRIPE)])

    return k


def _c1_body(p_ref, h_ref, root_ref, bias_ref, t_ref, st_ref):
    i = pl.program_id(0)
    p0 = p_ref[0]
    p1 = p_ref[1]
    s = p0[:, 0:HID] + p1[:, 0:HID]
    cnt = p0[:, HID:HID + 1] + p1[:, HID:HID + 1]
    agg = s * _recip(jnp.maximum(cnt, 1.0))
    t = agg + _bfdot(h_ref[...], root_ref[...]) + bias_ref[...]
    t_ref[...] = t
    blk = jnp.concatenate(
        [jnp.sum(t, axis=0, keepdims=True), jnp.sum(t * t, axis=0, keepdims=True)],
        axis=0)

    @pl.when(i == 0)
    def _():
        st_ref[...] = blk

    @pl.when(i > 0)
    def _():
        st_ref[...] = st_ref[...] + blk


def _c1_call(P, h, root, bias):
    in_c = h.shape[-1]
    return pl.pallas_call(
        _c1_body,
        grid=(N_NODES // BN_BLK,),
        in_specs=[
            pl.BlockSpec((NC, BN_BLK, 32), lambda i: (0, i, 0)),
            pl.BlockSpec((BN_BLK, in_c), lambda i: (i, 0)),
            pl.BlockSpec((in_c, HID), lambda i: (0, 0)),
            pl.BlockSpec((1, HID), lambda i: (0, 0)),
        ],
        out_specs=[
            pl.BlockSpec((BN_BLK, HID), lambda i: (i, 0)),
            pl.BlockSpec((2, HID), lambda i: (0, 0)),
        ],
        out_shape=[
            jax.ShapeDtypeStruct((N_NODES, HID), jnp.float32),
            jax.ShapeDtypeStruct((2, HID), jnp.float32),
        ],
    )(P, h, root, bias)


def _c2_body_res(t_ref, st_ref, g_ref, b_ref, hp_ref, o_ref):
    mean = st_ref[0:1, :] * (1.0 / N_NODES)
    var = st_ref[1:2, :] * (1.0 / N_NODES) - mean * mean
    xn = (t_ref[...] - mean) * _rsqrt_acc(var + 1e-5) * g_ref[...] + b_ref[...]
    o_ref[...] = jnp.maximum(xn, 0.0) + hp_ref[...]


def _c2_body_nores(t_ref, st_ref, g_ref, b_ref, o_ref):
    mean = st_ref[0:1, :] * (1.0 / N_NODES)
    var = st_ref[1:2, :] * (1.0 / N_NODES) - mean * mean
    xn = (t_ref[...] - mean) * _rsqrt_acc(var + 1e-5) * g_ref[...] + b_ref[...]
    o_ref[...] = jnp.maximum(xn, 0.0)


def _c2_call(residual, t, st, g, b, h_prev):
    in_specs = [
        pl.BlockSpec((BN_BLK, HID), lambda i: (i, 0)),
        pl.BlockSpec((2, HID), lambda i: (0, 0)),
        pl.BlockSpec((1, HID), lambda i: (0, 0)),
        pl.BlockSpec((1, HID), lambda i: (0, 0)),
    ]
    args = [t, st, g, b]
    if residual:
        in_specs.append(pl.BlockSpec((BN_BLK, HID), lambda i: (i, 0)))
        args.append(h_prev)
    return pl.pallas_call(
        _c2_body_res if residual else _c2_body_nores,
        grid=(N_NODES // BN_BLK,),
        in_specs=in_specs,
        out_specs=pl.BlockSpec((BN_BLK, HID), lambda i: (i, 0)),
        out_shape=jax.ShapeDtypeStruct((N_NODES, HID), jnp.float32),
    )(*args)


def _pool_body(h_ref, b_ref, out_ref, acc_s, cnt_s):
    i = pl.program_id(0)
    nsteps = pl.num_programs(0)
    bids = b_ref[:, 0]
    oh = (bids[:, None] == lax.broadcasted_iota(jnp.int32, (1, N_GRAPHS), 1)
          ).astype(jnp.float32)
    ssum = lax.dot_general(oh, h_ref[...], (((0,), (0,)), ((), ())),
                           preferred_element_type=jnp.float32,
                           precision=lax.Precision.HIGHEST)
    scnt = jnp.sum(oh, axis=0, keepdims=True)

    @pl.when(i == 0)
    def _():
        acc_s[...] = ssum
        cnt_s[...] = scnt

    @pl.when(i > 0)
    def _():
        acc_s[...] = acc_s[...] + ssum
        cnt_s[...] = cnt_s[...] + scnt

    @pl.when(i == nsteps - 1)
    def _():
        out_ref[...] = acc_s[...] * _recip(
            jnp.maximum(jnp.reshape(cnt_s[...], (N_GRAPHS, 1)), 1.0))


def _pool_call(h, batch2d):
    return pl.pallas_call(
        _pool_body,
        grid=(N_NODES // BN_BLK,),
        in_specs=[
            pl.BlockSpec((BN_BLK, HID), lambda i: (i, 0)),
            pl.BlockSpec((BN_BLK, 1), lambda i: (i, 0)),
        ],
        out_specs=pl.BlockSpec((N_GRAPHS, HID), lambda i: (0, 0)),
        out_shape=jax.ShapeDtypeStruct((N_GRAPHS, HID), jnp.float32),
        scratch_shapes=[
            pltpu.VMEM((N_GRAPHS, HID), jnp.float32),
            pltpu.VMEM((1, N_GRAPHS), jnp.float32),
        ],
    )(h, batch2d)


def kernel(x, edge_index, batch, edge_attr, params):
    E = edge_attr.shape[0]
    npad = E_PAD - E
    src_p = jnp.concatenate(
        [edge_index[0], jnp.zeros((npad,), jnp.int32)]).reshape(E_PAD // BE, BE)
    dst_p = jnp.concatenate(
        [edge_index[1], jnp.full((npad,), N_NODES, jnp.int32)]).reshape(E_PAD // BE, BE)
    ea_p = jnp.concatenate(
        [edge_attr, jnp.zeros((npad, EDGE_DIM), jnp.float32)]
    ).astype(jnp.bfloat16).astype(jnp.float32).reshape(-1)
    zeros_stripe = jnp.zeros((STRIPE, 32), jnp.float32)

    h = x
    for l, p in enumerate(params["convs"]):
        in_c = h.shape[-1]
        T = p["ew2"].reshape(HID, in_c, HID).transpose(1, 0, 2).reshape(in_c, HID * HID)
        C = jnp.concatenate([T, p["eb2"].reshape(in_c, HID)], axis=1)
        Z = _z_call(h, C)
        ew1_bf = p["ew1"].astype(jnp.bfloat16).astype(jnp.float32)
        P = _sc_call()(Z, ea_p, ew1_bf, p["eb1"][None, :], src_p, dst_p,
                       zeros_stripe)
        t, st = _c1_call(P, h, p["root"], p["bias"][None, :])
        h = _c2_call(l > 0, t, st, p["bn_g"][None, :], p["bn_b"][None, :], h)

    pooled = _pool_call(h, batch[:, None])
    h1 = jax.nn.relu(pooled @ params["lin1_w"] + params["lin1_b"])
    return h1 @ params["lin2_w"] + params["lin2_b"]
